# baseline (speedup 1.0000x reference)
import jax
import jax.numpy as jnp
from jax.experimental import pallas as pl
from jax.experimental.pallas import tpu as pltpu

_B, _S, _D = 16, 1024, 256
_ROWS = _B * _S
_K = 8
_CH = _ROWS // _K


def _copy_kernel(x_hbm, o_hbm, buf, insems, outsems):
    for k in range(_K):
        pltpu.make_async_copy(
            x_hbm.at[pl.ds(k * _CH, _CH), :], buf.at[k], insems.at[k]
        ).start()
    for k in range(_K):
        pltpu.make_async_copy(
            x_hbm.at[pl.ds(k * _CH, _CH), :], buf.at[k], insems.at[k]
        ).wait()
        pltpu.make_async_copy(
            buf.at[k], o_hbm.at[pl.ds(k * _CH, _CH), :], outsems.at[k]
        ).start()
    for k in range(_K):
        pltpu.make_async_copy(
            buf.at[k], o_hbm.at[pl.ds(k * _CH, _CH), :], outsems.at[k]
        ).wait()


def kernel(x, W):
    del W
    flat = x.reshape(_ROWS, _D)
    out = pl.pallas_call(
        _copy_kernel,
        in_specs=[pl.BlockSpec(memory_space=pltpu.MemorySpace.HBM)],
        out_specs=pl.BlockSpec(memory_space=pltpu.MemorySpace.HBM),
        out_shape=jax.ShapeDtypeStruct((_ROWS, _D), x.dtype),
        scratch_shapes=[
            pltpu.VMEM((_K, _CH, _D), x.dtype),
            pltpu.SemaphoreType.DMA((_K,)),
            pltpu.SemaphoreType.DMA((_K,)),
        ],
    )(flat)
    return out.reshape(x.shape)


# speedup vs baseline: 1.0104x; 1.0104x over previous
import jax
import jax.numpy as jnp
from jax.experimental import pallas as pl
from jax.experimental.pallas import tpu as pltpu

_B, _S, _D = 16, 1024, 256
_ROWS = _B * _S
_K = 4
_CH = _ROWS // _K


def _copy_kernel(x_hbm, o_hbm, buf, insems, outsems):
    for k in range(_K):
        pltpu.make_async_copy(
            x_hbm.at[pl.ds(k * _CH, _CH), :], buf.at[k], insems.at[k]
        ).start()
    for k in range(_K):
        pltpu.make_async_copy(
            x_hbm.at[pl.ds(k * _CH, _CH), :], buf.at[k], insems.at[k]
        ).wait()
        pltpu.make_async_copy(
            buf.at[k], o_hbm.at[pl.ds(k * _CH, _CH), :], outsems.at[k]
        ).start()
    for k in range(_K):
        pltpu.make_async_copy(
            buf.at[k], o_hbm.at[pl.ds(k * _CH, _CH), :], outsems.at[k]
        ).wait()


def kernel(x, W):
    del W
    flat = x.reshape(_ROWS, _D)
    out = pl.pallas_call(
        _copy_kernel,
        in_specs=[pl.BlockSpec(memory_space=pltpu.MemorySpace.HBM)],
        out_specs=pl.BlockSpec(memory_space=pltpu.MemorySpace.HBM),
        out_shape=jax.ShapeDtypeStruct((_ROWS, _D), x.dtype),
        scratch_shapes=[
            pltpu.VMEM((_K, _CH, _D), x.dtype),
            pltpu.SemaphoreType.DMA((_K,)),
            pltpu.SemaphoreType.DMA((_K,)),
        ],
    )(flat)
    return out.reshape(x.shape)


# speedup vs baseline: 1.0559x; 1.0451x over previous
import jax
import jax.numpy as jnp
from jax.experimental import pallas as pl
from jax.experimental.pallas import tpu as pltpu

_B, _S, _D = 16, 1024, 256
_ROWS = _B * _S
_SIZES = (1024, 4096, 6144, 4096, 1024)
_OFFS = tuple(sum(_SIZES[:k]) for k in range(len(_SIZES)))
_K = len(_SIZES)


def _copy_kernel(x_hbm, o_hbm, buf, insems, outsems):
    for k in range(_K):
        pltpu.make_async_copy(
            x_hbm.at[pl.ds(_OFFS[k], _SIZES[k]), :],
            buf.at[pl.ds(_OFFS[k], _SIZES[k]), :],
            insems.at[k],
        ).start()
    for k in range(_K):
        pltpu.make_async_copy(
            x_hbm.at[pl.ds(_OFFS[k], _SIZES[k]), :],
            buf.at[pl.ds(_OFFS[k], _SIZES[k]), :],
            insems.at[k],
        ).wait()
        pltpu.make_async_copy(
            buf.at[pl.ds(_OFFS[k], _SIZES[k]), :],
            o_hbm.at[pl.ds(_OFFS[k], _SIZES[k]), :],
            outsems.at[k],
        ).start()
    for k in range(_K):
        pltpu.make_async_copy(
            buf.at[pl.ds(_OFFS[k], _SIZES[k]), :],
            o_hbm.at[pl.ds(_OFFS[k], _SIZES[k]), :],
            outsems.at[k],
        ).wait()


def kernel(x, W):
    del W
    flat = x.reshape(_ROWS, _D)
    out = pl.pallas_call(
        _copy_kernel,
        in_specs=[pl.BlockSpec(memory_space=pltpu.MemorySpace.HBM)],
        out_specs=pl.BlockSpec(memory_space=pltpu.MemorySpace.HBM),
        out_shape=jax.ShapeDtypeStruct((_ROWS, _D), x.dtype),
        scratch_shapes=[
            pltpu.VMEM((_ROWS, _D), x.dtype),
            pltpu.SemaphoreType.DMA((_K,)),
            pltpu.SemaphoreType.DMA((_K,)),
        ],
    )(flat)
    return out.reshape(x.shape)


# speedup vs baseline: 1.0591x; 1.0030x over previous
import jax
import jax.numpy as jnp
from jax.experimental import pallas as pl
from jax.experimental.pallas import tpu as pltpu

_B, _S, _D = 16, 1024, 256
_ROWS = _B * _S
_SIZES = (256, 512, 1024, 2048, 4352, 4352, 2048, 1024, 512, 256)
_OFFS = tuple(sum(_SIZES[:k]) for k in range(len(_SIZES)))
_K = len(_SIZES)


def _copy_kernel(x_hbm, o_hbm, buf, insems, outsems):
    for k in range(_K):
        pltpu.make_async_copy(
            x_hbm.at[pl.ds(_OFFS[k], _SIZES[k]), :],
            buf.at[pl.ds(_OFFS[k], _SIZES[k]), :],
            insems.at[k],
        ).start()
    for k in range(_K):
        pltpu.make_async_copy(
            x_hbm.at[pl.ds(_OFFS[k], _SIZES[k]), :],
            buf.at[pl.ds(_OFFS[k], _SIZES[k]), :],
            insems.at[k],
        ).wait()
        pltpu.make_async_copy(
            buf.at[pl.ds(_OFFS[k], _SIZES[k]), :],
            o_hbm.at[pl.ds(_OFFS[k], _SIZES[k]), :],
            outsems.at[k],
        ).start()
    for k in range(_K):
        pltpu.make_async_copy(
            buf.at[pl.ds(_OFFS[k], _SIZES[k]), :],
            o_hbm.at[pl.ds(_OFFS[k], _SIZES[k]), :],
            outsems.at[k],
        ).wait()


def kernel(x, W):
    del W
    flat = x.reshape(_ROWS, _D)
    out = pl.pallas_call(
        _copy_kernel,
        in_specs=[pl.BlockSpec(memory_space=pltpu.MemorySpace.HBM)],
        out_specs=pl.BlockSpec(memory_space=pltpu.MemorySpace.HBM),
        out_shape=jax.ShapeDtypeStruct((_ROWS, _D), x.dtype),
        scratch_shapes=[
            pltpu.VMEM((_ROWS, _D), x.dtype),
            pltpu.SemaphoreType.DMA((_K,)),
            pltpu.SemaphoreType.DMA((_K,)),
        ],
    )(flat)
    return out.reshape(x.shape)
